# Initial kernel scaffold; baseline (speedup 1.0000x reference)
import jax
import jax.numpy as jnp
from jax.experimental import pallas as pl


def kernel(x, edge_index, W1, b1, W2, b2, W3, b3):
    raise NotImplementedError("write your pallas kernel here")


# speedup vs baseline: 1.2379x; 1.2379x over previous
import functools

import jax
import jax.numpy as jnp
from jax import lax
from jax.experimental import pallas as pl
from jax.experimental.pallas import tpu as pltpu
from jax.experimental.pallas import tpu_sc as plsc

L = 16
NW = 32

_GDN = lax.GatherDimensionNumbers(
    offset_dims=(), collapsed_slice_dims=(0,), start_index_map=(0,)
)


def _lane_shuffle(x, idx2d):
    return lax.gather(x, idx2d, _GDN, (1,),
                      mode=lax.GatherScatterMode.PROMISE_IN_BOUNDS)


def _pre_body(d, x_ref, w1_ref, b1_ref, w_ref):
    xb = x_ref[...]
    w_ref[0] = (
        jnp.dot(xb, w1_ref[:d, :], preferred_element_type=jnp.float32)
        + b1_ref[...]
    )
    w_ref[1] = jnp.dot(xb, w1_ref[d:, :], preferred_element_type=jnp.float32)
    w_ref[2] = xb


def _precompute(x, W1, b1):
    n, d = x.shape
    bn = 1000 if n % 1000 == 0 else n
    grid = n // bn
    return pl.pallas_call(
        functools.partial(_pre_body, d),
        grid=(grid,),
        in_specs=[
            pl.BlockSpec((bn, d), lambda i: (i, 0)),
            pl.BlockSpec((2 * d, d), lambda i: (0, 0)),
            pl.BlockSpec((1, d), lambda i: (0, 0)),
        ],
        out_specs=pl.BlockSpec((3, bn, d), lambda i: (0, i, 0)),
        out_shape=jax.ShapeDtypeStruct((3, n, d), jnp.float32),
    )(x, W1, b1.reshape(1, d))


def _mask_body(nchunks, c, d,
               eb_h, w_h, w2_h, b2_h, m_h,
               idxuv, uvrows, mbuf, w2v, b2v, sem1):
    nd = d // L
    cid = lax.axis_index("c")
    sid = lax.axis_index("s")
    wid = sid * 2 + cid

    pltpu.sync_copy(w2_h, w2v)
    pltpu.sync_copy(b2_h, b2v)
    w2regs = [w2v[pl.ds(k * L, L)] for k in range(nd)]
    b2vec = b2v[...]
    zeros = jnp.zeros((L,), jnp.float32)
    lanes = lax.iota(jnp.int32, L)
    perms = [(lanes ^ sh)[:, None] for sh in (8, 4, 2, 1)]

    def mchunk(k, carry):
        pltpu.sync_copy(eb_h.at[wid, k, pl.ds(0, 2 * c)], idxuv)
        pltpu.async_copy(w_h.at[idxuv], uvrows, sem1).wait()

        def edge(e, mgrp):
            acc = zeros
            for kk in range(nd):
                uu = uvrows[e, pl.ds(kk * L, L)]
                vv = uvrows[c + e, pl.ds(kk * L, L)]
                acc = acc + jnp.maximum(uu + vv, 0.0) * w2regs[kk]
            for pp in perms:
                acc = acc + _lane_shuffle(acc, pp)
            mv = 1.0 / (1.0 + jnp.exp(-(acc + b2vec)))
            j = jnp.bitwise_and(e, L - 1)
            mgrp = jnp.where(lanes == j, mv, mgrp)

            @pl.when(j == L - 1)
            def _():
                mbuf[pl.ds(k * c + e - (L - 1), L)] = mgrp

            return mgrp

        lax.fori_loop(0, c, edge, zeros)
        return carry

    lax.fori_loop(0, nchunks, mchunk, 0)
    pltpu.sync_copy(mbuf, m_h.at[wid])


def _mask_pass(eb, W, W2f, b2v):
    d = W.shape[1]
    nw, nchunks, c3 = eb.shape
    c = c3 // 3
    mesh = plsc.VectorSubcoreMesh(core_axis_name="c", subcore_axis_name="s",
                                  num_cores=2, num_subcores=16)
    body = functools.partial(_mask_body, nchunks, c, d)
    f = pl.kernel(
        body,
        out_type=jax.ShapeDtypeStruct((nw, nchunks * c), jnp.float32),
        mesh=mesh,
        scratch_types=[
            pltpu.VMEM((2 * c,), jnp.int32),
            pltpu.VMEM((2 * c, d), jnp.float32),
            pltpu.VMEM((nchunks * c,), jnp.float32),
            pltpu.VMEM((d,), jnp.float32),
            pltpu.VMEM((L,), jnp.float32),
            pltpu.SemaphoreType.DMA,
        ],
    )
    return f(eb, W, W2f, b2v)


def _scatter_body(nchunks, c, n_nodes, d, ar,
                  eb_h, w_h, m_h,
                  outa_h, outm_h,
                  idxr, idxmod, xrows, stga, stgm, mbuf,
                  acca, accm, sem1):
    nd = d // L
    q = ar - L
    npass = -(-n_nodes // q)
    dump = ar - 1
    pta = ar // 16
    cid = lax.axis_index("c")
    sid = lax.axis_index("s")
    wid = sid * 2 + cid

    zeros = jnp.zeros((L,), jnp.float32)
    ones = jnp.full((L,), 1.0, jnp.float32)
    lanes = lax.iota(jnp.int32, L)
    lane0 = lanes == 0
    bidx = [jnp.full((L, 1), j, jnp.int32) for j in range(L)]

    pltpu.sync_copy(m_h.at[wid], mbuf)

    def zrow(i, carry):
        for k in range(nd):
            stga[i, pl.ds(k * L, L)] = zeros
        stgm[i, :] = zeros
        return carry

    lax.fori_loop(0, c, zrow, 0)
    base_a = sid * pta
    nz = pta // c

    for p in range(npass):
        base_p = p * q
        qp = min(q, n_nodes - base_p)

        def zcpy(k, carry):
            pltpu.sync_copy(stga, acca.at[pl.ds(base_a + k * c, c)])
            pltpu.sync_copy(stgm, accm.at[pl.ds(base_a + k * c, c)])
            return carry

        lax.fori_loop(0, nz, zcpy, 0)
        plsc.subcore_barrier()

        def chunk(k, carry):
            pltpu.sync_copy(eb_h.at[wid, k, pl.ds(2 * c, c)], idxr)
            pltpu.async_copy(w_h.at[idxr], xrows, sem1).wait()
            pltpu.sync_copy(eb_h.at[wid, k, pl.ds(0, c)], idxr)

            def grp(g, carry2):
                mgrp = mbuf[pl.ds(k * c + g * L, L)]
                for j in range(L):
                    e = g * L + j
                    mj = _lane_shuffle(mgrp, bidx[j])
                    for kk in range(nd):
                        xv = xrows[e, pl.ds(kk * L, L)]
                        stga[e, pl.ds(kk * L, L)] = mj * xv
                    stgm[e, :] = jnp.where(
                        lane0, mj, jnp.where(lanes == 1, ones, zeros))
                return carry2

            lax.fori_loop(0, c // L, grp, 0)

            def remap(g, carry2):
                v = idxr[pl.ds(g * L, L)]
                rel = v - base_p
                inb = (rel >= 0) & (rel < q)
                idxmod[pl.ds(g * L, L)] = jnp.where(inb, rel, dump)
                return carry2

            lax.fori_loop(0, c // L, remap, 0)
            pltpu.sync_copy(stga, acca.at[idxmod], add=True)
            pltpu.sync_copy(stgm, accm.at[idxmod], add=True)
            return carry

        lax.fori_loop(0, nchunks, chunk, 0)
        plsc.subcore_barrier()

        full, remw = divmod(qp, pta)

        @pl.when(sid < full)
        def _():
            pltpu.sync_copy(acca.at[pl.ds(base_a, pta)],
                            outa_h.at[cid, pl.ds(base_p + base_a, pta)])
            pltpu.sync_copy(accm.at[pl.ds(base_a, pta)],
                            outm_h.at[cid, pl.ds(base_p + base_a, pta)])

        if remw:
            @pl.when(sid == full)
            def _():
                pltpu.sync_copy(
                    acca.at[pl.ds(full * pta, remw)],
                    outa_h.at[cid, pl.ds(base_p + full * pta, remw)])
                pltpu.sync_copy(
                    accm.at[pl.ds(full * pta, remw)],
                    outm_h.at[cid, pl.ds(base_p + full * pta, remw)])


def _scatter_pass(eb, W, m):
    n3, d = W.shape
    n = n3 // 3
    nw, nchunks, c3 = eb.shape
    c = c3 // 3
    ar = 80 * c
    mesh = plsc.VectorSubcoreMesh(core_axis_name="c", subcore_axis_name="s",
                                  num_cores=2, num_subcores=16)
    body = functools.partial(_scatter_body, nchunks, c, n, d, ar)
    f = pl.kernel(
        body,
        out_type=[
            jax.ShapeDtypeStruct((2, n, d), jnp.float32),
            jax.ShapeDtypeStruct((2, n, L), jnp.float32),
        ],
        mesh=mesh,
        scratch_types=[
            pltpu.VMEM((c,), jnp.int32),
            pltpu.VMEM((c,), jnp.int32),
            pltpu.VMEM((c, d), jnp.float32),
            pltpu.VMEM((c, d), jnp.float32),
            pltpu.VMEM((c, L), jnp.float32),
            pltpu.VMEM((nchunks * c,), jnp.float32),
            pltpu.VMEM_SHARED((ar, d), jnp.float32),
            pltpu.VMEM_SHARED((ar, L), jnp.float32),
            pltpu.SemaphoreType.DMA,
        ],
    )
    return f(eb, W, m)


def _fin_body(x_ref, a_ref, m_ref, w3_ref, b3_ref, o_ref):
    xb = x_ref[...]
    agg = a_ref[0] + a_ref[1]
    meta = m_ref[0] + m_ref[1]
    msum = meta[:, 0:1]
    cnt = meta[:, 1:2]
    oe = jnp.where(cnt > 0.0, 0.5 * xb + 0.5 * agg / (msum + 1e-9), xb)
    o_ref[...] = jnp.maximum(
        jnp.dot(oe, w3_ref[...], preferred_element_type=jnp.float32)
        + b3_ref[...],
        0.0,
    )


def _finalize(x, outa, outm, W3, b3):
    n, d = x.shape
    out = W3.shape[1]
    bn = 1000 if n % 1000 == 0 else n
    grid = n // bn
    return pl.pallas_call(
        _fin_body,
        grid=(grid,),
        in_specs=[
            pl.BlockSpec((bn, d), lambda i: (i, 0)),
            pl.BlockSpec((2, bn, d), lambda i: (0, i, 0)),
            pl.BlockSpec((2, bn, L), lambda i: (0, i, 0)),
            pl.BlockSpec((d, out), lambda i: (0, 0)),
            pl.BlockSpec((1, out), lambda i: (0, 0)),
        ],
        out_specs=pl.BlockSpec((bn, out), lambda i: (i, 0)),
        out_shape=jax.ShapeDtypeStruct((n, out), jnp.float32),
    )(x, outa, outm, W3, b3.reshape(1, out))


def kernel(x, edge_index, W1, b1, W2, b2, W3, b3):
    n, d = x.shape
    e = edge_index.shape[1]
    c = 64
    per_w = -(-e // (NW * c)) * c
    nchunks = per_w // c
    epad = NW * per_w - e

    Wt = _precompute(x, W1, b1).reshape(3 * n, d)
    row = jnp.concatenate(
        [edge_index[0], jnp.full((epad,), n, jnp.int32)]).reshape(
            NW, nchunks, c)
    col = jnp.concatenate(
        [edge_index[1], jnp.zeros((epad,), jnp.int32)]).reshape(
            NW, nchunks, c)
    eb = jnp.concatenate([row, n + col, 2 * n + col], axis=2)
    b2v = jnp.broadcast_to(b2.reshape(1), (L,)).astype(jnp.float32)
    m = _mask_pass(eb, Wt, W2.reshape(d), b2v)
    mm = m.reshape(-1)[:e]
    row0 = edge_index[0]
    col0 = edge_index[1]
    denom = jax.ops.segment_sum(mm, row0, num_segments=n) + 1e-09
    w = mm / jnp.take(denom, row0, axis=0)
    agg = jax.ops.segment_sum(w[:, None] * jnp.take(x, col0, axis=0), row0,
                              num_segments=n)
    deg = jax.ops.segment_sum(jnp.ones_like(mm), row0, num_segments=n)
    oe = jnp.where(deg[:, None] > 0, 0.5 * x + 0.5 * agg, x)
    return jax.nn.relu(oe @ W3 + b3)


# speedup vs baseline: 1.2385x; 1.0005x over previous
import functools

import jax
import jax.numpy as jnp
from jax import lax
from jax.experimental import pallas as pl
from jax.experimental.pallas import tpu as pltpu
from jax.experimental.pallas import tpu_sc as plsc

L = 16
NW = 32

_GDN = lax.GatherDimensionNumbers(
    offset_dims=(), collapsed_slice_dims=(0,), start_index_map=(0,)
)


def _lane_shuffle(x, idx2d):
    return lax.gather(x, idx2d, _GDN, (1,),
                      mode=lax.GatherScatterMode.PROMISE_IN_BOUNDS)


def _pre_body(d, x_ref, w1_ref, b1_ref, w_ref):
    xb = x_ref[...]
    w_ref[0] = (
        jnp.dot(xb, w1_ref[:d, :], preferred_element_type=jnp.float32)
        + b1_ref[...]
    )
    w_ref[1] = jnp.dot(xb, w1_ref[d:, :], preferred_element_type=jnp.float32)
    w_ref[2] = xb


def _precompute(x, W1, b1):
    n, d = x.shape
    bn = 1000 if n % 1000 == 0 else n
    grid = n // bn
    return pl.pallas_call(
        functools.partial(_pre_body, d),
        grid=(grid,),
        in_specs=[
            pl.BlockSpec((bn, d), lambda i: (i, 0)),
            pl.BlockSpec((2 * d, d), lambda i: (0, 0)),
            pl.BlockSpec((1, d), lambda i: (0, 0)),
        ],
        out_specs=pl.BlockSpec((3, bn, d), lambda i: (0, i, 0)),
        out_shape=jax.ShapeDtypeStruct((3, n, d), jnp.float32),
    )(x, W1, b1.reshape(1, d))


def _mask_body(nchunks, c, d,
               eb_h, w_h, w2_h, b2_h, m_h,
               idxuv, uvrows, mbuf, w2v, b2v, sem1):
    nd = d // L
    cid = lax.axis_index("c")
    sid = lax.axis_index("s")
    wid = sid * 2 + cid

    pltpu.sync_copy(w2_h, w2v)
    pltpu.sync_copy(b2_h, b2v)
    w2regs = [w2v[pl.ds(k * L, L)] for k in range(nd)]
    b2vec = b2v[...]
    zeros = jnp.zeros((L,), jnp.float32)
    lanes = lax.iota(jnp.int32, L)
    perms = [(lanes ^ sh)[:, None] for sh in (8, 4, 2, 1)]

    def mchunk(k, carry):
        pltpu.sync_copy(eb_h.at[wid, k, pl.ds(0, 2 * c)], idxuv)
        pltpu.async_copy(w_h.at[idxuv], uvrows, sem1).wait()

        def edge(e, mgrp):
            acc = zeros
            for kk in range(nd):
                uu = uvrows[e, pl.ds(kk * L, L)]
                vv = uvrows[c + e, pl.ds(kk * L, L)]
                acc = acc + jnp.maximum(uu + vv, 0.0) * w2regs[kk]
            for pp in perms:
                acc = acc + _lane_shuffle(acc, pp)
            mv = 1.0 / (1.0 + jnp.exp(-(acc + b2vec)))
            j = jnp.bitwise_and(e, L - 1)
            mgrp = jnp.where(lanes == j, mv, mgrp)

            @pl.when(j == L - 1)
            def _():
                mbuf[pl.ds(k * c + e - (L - 1), L)] = mgrp

            return mgrp

        lax.fori_loop(0, c, edge, zeros)
        return carry

    lax.fori_loop(0, nchunks, mchunk, 0)
    pltpu.sync_copy(mbuf, m_h.at[wid])


def _mask_pass(eb, W, W2f, b2v):
    d = W.shape[1]
    nw, nchunks, c3 = eb.shape
    c = c3 // 3
    mesh = plsc.VectorSubcoreMesh(core_axis_name="c", subcore_axis_name="s",
                                  num_cores=2, num_subcores=16)
    body = functools.partial(_mask_body, nchunks, c, d)
    f = pl.kernel(
        body,
        out_type=jax.ShapeDtypeStruct((nw, nchunks * c), jnp.float32),
        mesh=mesh,
        scratch_types=[
            pltpu.VMEM((2 * c,), jnp.int32),
            pltpu.VMEM((2 * c, d), jnp.float32),
            pltpu.VMEM((nchunks * c,), jnp.float32),
            pltpu.VMEM((d,), jnp.float32),
            pltpu.VMEM((L,), jnp.float32),
            pltpu.SemaphoreType.DMA,
        ],
    )
    return f(eb, W, W2f, b2v)


def kernel(x, edge_index, W1, b1, W2, b2, W3, b3):
    n, d = x.shape
    e = edge_index.shape[1]
    c = 64
    per_w = -(-e // (NW * c)) * c
    nchunks = per_w // c
    epad = NW * per_w - e

    Wt = _precompute(x, W1, b1).reshape(3 * n, d)
    row = jnp.concatenate(
        [edge_index[0], jnp.full((epad,), n, jnp.int32)]).reshape(
            NW, nchunks, c)
    col = jnp.concatenate(
        [edge_index[1], jnp.zeros((epad,), jnp.int32)]).reshape(
            NW, nchunks, c)
    eb = jnp.concatenate([row, n + col, 2 * n + col], axis=2)
    b2v = jnp.broadcast_to(b2.reshape(1), (L,)).astype(jnp.float32)
    m = _mask_pass(eb, Wt, W2.reshape(d), b2v)
    mm = m.reshape(-1)[:e]
    row0 = edge_index[0]
    col0 = edge_index[1]
    denom = jax.ops.segment_sum(mm, row0, num_segments=n) + 1e-09
    w = mm / jnp.take(denom, row0, axis=0)
    agg = jax.ops.segment_sum(w[:, None] * jnp.take(x, col0, axis=0), row0,
                              num_segments=n)
    deg = jax.ops.segment_sum(jnp.ones_like(mm), row0, num_segments=n)
    oe = jnp.where(deg[:, None] > 0, 0.5 * x + 0.5 * agg, x)
    return jax.nn.relu(oe @ W3 + b3)
